# baseline (speedup 1.0000x reference)
import functools

import jax
import jax.numpy as jnp
from jax import lax
from jax.experimental import pallas as pl
from jax.experimental.pallas import tpu as pltpu
from jax.experimental.pallas import tpu_sc as plsc


def _sc_embed_gather(ids, table):
    bsz, seq = ids.shape
    n_tokens = bsz * seq
    d_model = table.shape[1]

    info = plsc.get_sparse_core_info()
    nc, ns = info.num_cores, info.num_subcores
    nw = nc * ns
    n_per_w = n_tokens // nw
    w_per_row = seq // n_per_w
    chunk = 32
    nbuf = 2
    la = 1
    n_chunks = n_per_w // chunk
    assert seq % n_per_w == 0 and n_per_w % chunk == 0
    assert n_chunks % nbuf == 0 and 0 < la < nbuf
    assert nbuf * chunk * d_model * 4 <= 500 * 1024

    mesh = plsc.VectorSubcoreMesh(core_axis_name="c", subcore_axis_name="s")

    @functools.partial(
        pl.kernel,
        out_type=jax.ShapeDtypeStruct((n_tokens, d_model), jnp.float32),
        mesh=mesh,
        scratch_types=[
            pltpu.VMEM((n_per_w,), jnp.int32),
            pltpu.VMEM((nbuf, chunk, d_model), jnp.float32),
            pltpu.SemaphoreType.DMA,
            pltpu.SemaphoreType.DMA,
        ],
    )
    def k(ids_hbm, table_hbm, out_hbm, idx_v, rows_v, gsem, ssem):
        wid = lax.axis_index("s") * nc + lax.axis_index("c")
        base = wid * n_per_w
        pltpu.sync_copy(
            ids_hbm.at[wid // w_per_row,
                       pl.ds((wid % w_per_row) * n_per_w, n_per_w)],
            idx_v,
        )

        def _idx(j):
            return idx_v.at[pl.ds(pl.multiple_of(j * chunk, 8), chunk)]

        def gather_start(j, b):
            pltpu.async_copy(table_hbm.at[_idx(j)], rows_v.at[b], gsem)

        def gather_wait(j, b):
            pltpu.make_async_copy(
                table_hbm.at[_idx(j)], rows_v.at[b], gsem
            ).wait()

        def scat_start(j, b):
            pltpu.async_copy(
                rows_v.at[b], out_hbm.at[pl.ds(base + j * chunk, chunk)], ssem
            )

        def scat_wait(j, b):
            pltpu.make_async_copy(
                rows_v.at[b], out_hbm.at[pl.ds(base + j * chunk, chunk)], ssem
            ).wait()

        for j in range(la):
            gather_start(j, j % nbuf)

        def group(g, carry):
            for b in range(nbuf):
                j = nbuf * g + b
                jn = j + la

                @pl.when((jn < n_chunks) & (jn >= nbuf))
                def _():
                    scat_wait(jn - nbuf, jn % nbuf)

                @pl.when(jn < n_chunks)
                def _():
                    gather_start(jn, jn % nbuf)

                gather_wait(j, b)
                scat_start(j, b)
            return carry

        lax.fori_loop(0, n_chunks // nbuf, group, 0, unroll=False)
        for j in range(n_chunks - nbuf, n_chunks):
            scat_wait(j, j % nbuf)

    return k(ids, table)


def kernel(input_ids, attention_mask, embed_table):
    b, t = input_ids.shape
    hidden = _sc_embed_gather(input_ids.astype(jnp.int32), embed_table)
    hidden = hidden.reshape(b, t, embed_table.shape[1])
    position_ids = jnp.broadcast_to(
        jnp.arange(t, dtype=input_ids.dtype)[None, :], (b, t)
    )
    return (hidden, attention_mask, position_ids)


# speedup vs baseline: 1.0132x; 1.0132x over previous
import functools

import jax
import jax.numpy as jnp
from jax import lax
from jax.experimental import pallas as pl
from jax.experimental.pallas import tpu as pltpu
from jax.experimental.pallas import tpu_sc as plsc


def _sc_embed_gather(ids, table):
    bsz, seq = ids.shape
    n_tokens = bsz * seq
    d_model = table.shape[1]

    info = plsc.get_sparse_core_info()
    nc, ns = info.num_cores, info.num_subcores
    nw = nc * ns
    n_per_w = n_tokens // nw
    w_per_row = seq // n_per_w
    chunk = 16
    nbuf = 4
    la = 3
    n_chunks = n_per_w // chunk
    assert seq % n_per_w == 0 and n_per_w % chunk == 0
    assert n_chunks % nbuf == 0 and 0 < la < nbuf
    assert nbuf * chunk * d_model * 4 <= 500 * 1024

    mesh = plsc.VectorSubcoreMesh(core_axis_name="c", subcore_axis_name="s")

    @functools.partial(
        pl.kernel,
        out_type=(
            jax.ShapeDtypeStruct((n_tokens, d_model), jnp.float32),
            jax.ShapeDtypeStruct((bsz, seq), jnp.int32),
        ),
        mesh=mesh,
        scratch_types=[
            pltpu.VMEM((n_per_w,), jnp.int32),
            pltpu.VMEM((n_per_w,), jnp.int32),
            pltpu.VMEM((nbuf, chunk, d_model), jnp.float32),
            pltpu.SemaphoreType.DMA,
            pltpu.SemaphoreType.DMA,
            pltpu.SemaphoreType.DMA,
        ],
    )
    def k(ids_hbm, table_hbm, out_hbm, pos_hbm, idx_v, pos_v, rows_v,
          gsem, ssem, psem):
        wid = lax.axis_index("s") * nc + lax.axis_index("c")
        base = wid * n_per_w
        row = wid // w_per_row
        colbase = (wid % w_per_row) * n_per_w
        pltpu.sync_copy(
            ids_hbm.at[row, pl.ds(colbase, n_per_w)],
            idx_v,
        )

        piota = lax.iota(jnp.int32, 16)

        def fill(c, carry):
            off = pl.multiple_of(c * 16, 8)
            pos_v[pl.ds(off, 16)] = piota + (colbase + c * 16)
            return carry

        lax.fori_loop(0, n_per_w // 16, fill, 0)
        pltpu.async_copy(pos_v, pos_hbm.at[row, pl.ds(colbase, n_per_w)], psem)

        def _idx(j):
            return idx_v.at[pl.ds(pl.multiple_of(j * chunk, 8), chunk)]

        def gather_start(j, b):
            pltpu.async_copy(table_hbm.at[_idx(j)], rows_v.at[b], gsem)

        def gather_wait(j, b):
            pltpu.make_async_copy(
                table_hbm.at[_idx(j)], rows_v.at[b], gsem
            ).wait()

        def scat_start(j, b):
            pltpu.async_copy(
                rows_v.at[b], out_hbm.at[pl.ds(base + j * chunk, chunk)], ssem
            )

        def scat_wait(j, b):
            pltpu.make_async_copy(
                rows_v.at[b], out_hbm.at[pl.ds(base + j * chunk, chunk)], ssem
            ).wait()

        for j in range(la):
            gather_start(j, j % nbuf)

        def group(g, carry):
            for b in range(nbuf):
                j = nbuf * g + b
                jn = j + la

                @pl.when((jn < n_chunks) & (jn >= nbuf))
                def _():
                    scat_wait(jn - nbuf, jn % nbuf)

                @pl.when(jn < n_chunks)
                def _():
                    gather_start(jn, jn % nbuf)

                gather_wait(j, b)
                scat_start(j, b)
            return carry

        lax.fori_loop(0, n_chunks // nbuf, group, 0, unroll=False)
        for j in range(n_chunks - nbuf, n_chunks):
            scat_wait(j, j % nbuf)
        pltpu.make_async_copy(
            pos_v, pos_hbm.at[row, pl.ds(colbase, n_per_w)], psem
        ).wait()

    return k(ids, table)


def kernel(input_ids, attention_mask, embed_table):
    b, t = input_ids.shape
    hidden, position_ids = _sc_embed_gather(
        input_ids.astype(jnp.int32), embed_table
    )
    hidden = hidden.reshape(b, t, embed_table.shape[1])
    return (hidden, attention_mask, position_ids.astype(input_ids.dtype))


# speedup vs baseline: 1.0180x; 1.0047x over previous
import functools

import jax
import jax.numpy as jnp
from jax import lax
from jax.experimental import pallas as pl
from jax.experimental.pallas import tpu as pltpu
from jax.experimental.pallas import tpu_sc as plsc


def _sc_embed_gather(ids, table):
    bsz, seq = ids.shape
    n_tokens = bsz * seq
    d_model = table.shape[1]

    info = plsc.get_sparse_core_info()
    nc, ns = info.num_cores, info.num_subcores
    nw = nc * ns
    n_per_w = n_tokens // nw
    w_per_row = seq // n_per_w
    chunk = 16
    nbuf = 4
    la = 3
    n_chunks = n_per_w // chunk
    assert seq % n_per_w == 0 and n_per_w % chunk == 0
    assert n_chunks % nbuf == 0 and 0 < la < nbuf
    assert nbuf * chunk * d_model * 4 <= 500 * 1024

    mesh = plsc.VectorSubcoreMesh(core_axis_name="c", subcore_axis_name="s")

    @functools.partial(
        pl.kernel,
        out_type=(
            jax.ShapeDtypeStruct((n_tokens, d_model), jnp.float32),
            jax.ShapeDtypeStruct((bsz, seq), jnp.int32),
        ),
        mesh=mesh,
        scratch_types=[
            pltpu.VMEM((n_per_w,), jnp.int32),
            pltpu.VMEM((n_per_w,), jnp.int32),
            pltpu.VMEM((nbuf, chunk, d_model), jnp.float32),
            pltpu.SemaphoreType.DMA,
            pltpu.SemaphoreType.DMA,
            pltpu.SemaphoreType.DMA,
        ],
    )
    def k(ids_hbm, table_hbm, out_hbm, pos_hbm, idx_v, pos_v, rows_v,
          gsem, ssem, psem):
        wid = lax.axis_index("s") * nc + lax.axis_index("c")
        base = wid * n_per_w
        row = wid // w_per_row
        colbase = (wid % w_per_row) * n_per_w
        pltpu.sync_copy(
            ids_hbm.at[row, pl.ds(colbase, n_per_w)],
            idx_v,
        )


        def _idx(j):
            return idx_v.at[pl.ds(pl.multiple_of(j * chunk, 8), chunk)]

        def gather_start(j, b):
            pltpu.async_copy(table_hbm.at[_idx(j)], rows_v.at[b], gsem)

        def gather_wait(j, b):
            pltpu.make_async_copy(
                table_hbm.at[_idx(j)], rows_v.at[b], gsem
            ).wait()

        def scat_start(j, b):
            pltpu.async_copy(
                rows_v.at[b], out_hbm.at[pl.ds(base + j * chunk, chunk)], ssem
            )

        def scat_wait(j, b):
            pltpu.make_async_copy(
                rows_v.at[b], out_hbm.at[pl.ds(base + j * chunk, chunk)], ssem
            ).wait()

        for j in range(la):
            gather_start(j, j % nbuf)

        piota = lax.iota(jnp.int32, 16)

        def fill(c, carry):
            off = pl.multiple_of(c * 16, 8)
            pos_v[pl.ds(off, 16)] = piota + (colbase + c * 16)
            return carry

        lax.fori_loop(0, n_per_w // 16, fill, 0)
        pltpu.async_copy(pos_v, pos_hbm.at[row, pl.ds(colbase, n_per_w)], psem)

        def group(g, carry):
            for b in range(nbuf):
                j = nbuf * g + b
                jn = j + la

                @pl.when((jn < n_chunks) & (jn >= nbuf))
                def _():
                    scat_wait(jn - nbuf, jn % nbuf)

                @pl.when(jn < n_chunks)
                def _():
                    gather_start(jn, jn % nbuf)

                gather_wait(j, b)
                scat_start(j, b)
            return carry

        lax.fori_loop(0, n_chunks // nbuf, group, 0, unroll=False)
        for j in range(n_chunks - nbuf, n_chunks):
            scat_wait(j, j % nbuf)
        pltpu.make_async_copy(
            pos_v, pos_hbm.at[row, pl.ds(colbase, n_per_w)], psem
        ).wait()

    return k(ids, table)


def kernel(input_ids, attention_mask, embed_table):
    b, t = input_ids.shape
    hidden, position_ids = _sc_embed_gather(
        input_ids.astype(jnp.int32), embed_table
    )
    hidden = hidden.reshape(b, t, embed_table.shape[1])
    return (hidden, attention_mask, position_ids.astype(input_ids.dtype))
